# baseline (speedup 1.0000x reference)
import functools

import jax
import jax.numpy as jnp
from jax import lax
from jax.experimental import pallas as pl
from jax.experimental.pallas import tpu as pltpu
from jax.experimental.pallas import tpu_sc as plsc

EMBED = 64
BATCH = 16384
LAM = 1e-05

NC = 2
NS = 16
NW = NC * NS
PW = BATCH // NW
CH = 128
NCH = PW // CH


def _pack_body(x_ref, o_ref):
    t = jnp.transpose(x_ref[...], (1, 0))
    t3 = t.reshape(256, 2, 64)
    o_ref[...] = jnp.concatenate([t3[:, 0, :], t3[:, 1, :]], axis=1)


def _make_pack(n):
    grid = pl.cdiv(n, 512)
    return pl.pallas_call(
        _pack_body,
        grid=(grid,),
        in_specs=[pl.BlockSpec((EMBED, 512), lambda b: (0, b))],
        out_specs=pl.BlockSpec((256, 128), lambda b: (b, 0)),
        out_shape=jax.ShapeDtypeStruct((n // 2, 128), jnp.float32),
    )


def _sc_body(h_hbm, r_hbm, p_hbm, n_hbm, ent_hbm, rel_hbm,
             delta_hbm, l2_hbm,
             hidx, ridx, pidx, nidx,
             ghidx, gridx, gpidx, gnidx,
             hbuf, rbuf, pbuf, nbuf,
             dout, l2v, sem):
    wid = lax.axis_index("s") * NC + lax.axis_index("c")
    base = wid * PW

    l2 = jnp.zeros((16,), jnp.float32)
    for c in range(NCH):
        row0 = base + c * CH
        pltpu.sync_copy(h_hbm.at[pl.ds(row0, CH)], hidx.at[c])
        pltpu.sync_copy(r_hbm.at[pl.ds(row0, CH)], ridx.at[c])
        pltpu.sync_copy(p_hbm.at[pl.ds(row0, CH)], pidx.at[c])
        pltpu.sync_copy(n_hbm.at[pl.ds(row0, CH)], nidx.at[c])

        for j in range(CH // 16):
            sl = pl.ds(16 * j, 16)
            ghidx[c, sl] = lax.shift_right_logical(hidx[c, sl], 1)
            gridx[c, sl] = lax.shift_right_logical(ridx[c, sl], 1)
            gpidx[c, sl] = lax.shift_right_logical(pidx[c, sl], 1)
            gnidx[c, sl] = lax.shift_right_logical(nidx[c, sl], 1)

        cps = [
            pltpu.async_copy(ent_hbm.at[ghidx.at[c]], hbuf, sem),
            pltpu.async_copy(rel_hbm.at[gridx.at[c]], rbuf, sem),
            pltpu.async_copy(ent_hbm.at[gpidx.at[c]], pbuf, sem),
            pltpu.async_copy(ent_hbm.at[gnidx.at[c]], nbuf, sem),
        ]
        for cp in cps:
            cp.wait()

        def group_body(g, l2c):
            sl16 = pl.ds(16 * g, 16)
            he16 = hidx[c, sl16]
            re16 = ridx[c, sl16]
            pe16 = pidx[c, sl16]
            ne16 = nidx[c, sl16]
            for l in range(16):
                i = 16 * g + l
                oh = 64 * (he16[l] & 1)
                orr = 64 * (re16[l] & 1)
                op = 64 * (pe16[l] & 1)
                on = 64 * (ne16[l] & 1)
                dl = jnp.zeros((16,), jnp.float32)
                for d in range(EMBED // 16):
                    hv = hbuf[i, pl.ds(oh + 16 * d, 16)]
                    rv = rbuf[i, pl.ds(orr + 16 * d, 16)]
                    pv = pbuf[i, pl.ds(op + 16 * d, 16)]
                    nv = nbuf[i, pl.ds(on + 16 * d, 16)]
                    s = hv + rv
                    dp = s - pv
                    dn = s - nv
                    dl = dl + (dp * dp - dn * dn)
                    l2c = l2c + hv * hv + rv * rv + pv * pv + nv * nv
                dout[2 * g + l // 8, pl.ds(16 * (l % 8), 16)] = dl
            return l2c

        l2 = lax.fori_loop(0, CH // 16, group_body, l2)
        pltpu.sync_copy(dout, delta_hbm.at[pl.ds(wid * (PW // 8) + c * (CH // 8), CH // 8)])

    l2v[...] = l2
    pltpu.sync_copy(l2v, l2_hbm.at[wid // 8, pl.ds(16 * (wid % 8), 16)])


_sc_call = pl.kernel(
    _sc_body,
    out_type=[
        jax.ShapeDtypeStruct((BATCH // 8, 128), jnp.float32),
        jax.ShapeDtypeStruct((NW // 8, 128), jnp.float32),
    ],
    mesh=plsc.VectorSubcoreMesh(core_axis_name="c", subcore_axis_name="s"),
    scratch_types=[
        pltpu.VMEM((NCH, CH), jnp.int32),
        pltpu.VMEM((NCH, CH), jnp.int32),
        pltpu.VMEM((NCH, CH), jnp.int32),
        pltpu.VMEM((NCH, CH), jnp.int32),
        pltpu.VMEM((NCH, CH), jnp.int32),
        pltpu.VMEM((NCH, CH), jnp.int32),
        pltpu.VMEM((NCH, CH), jnp.int32),
        pltpu.VMEM((NCH, CH), jnp.int32),
        pltpu.VMEM((CH, 128), jnp.float32),
        pltpu.VMEM((CH, 128), jnp.float32),
        pltpu.VMEM((CH, 128), jnp.float32),
        pltpu.VMEM((CH, 128), jnp.float32),
        pltpu.VMEM((CH // 8, 128), jnp.float32),
        pltpu.VMEM((16,), jnp.float32),
        pltpu.SemaphoreType.DMA,
    ],
)


def _tc_body(x_ref, l2_ref, out_ref):
    x = x_ref[...]
    g = lax.broadcasted_iota(jnp.int32, (128, 8), 0) // 16
    c = lax.broadcasted_iota(jnp.int32, (128, 8), 1)
    m = (g == c).astype(jnp.float32)
    y = lax.dot_general(x, m, (((1,), (0,)), ((), ())),
                        preferred_element_type=jnp.float32)
    sp = jnp.maximum(y, 0.0) + jnp.log1p(jnp.exp(-jnp.abs(y)))
    l2tot = jnp.sum(l2_ref[...])
    loss = jnp.sum(sp) / BATCH + LAM * (l2tot / (2.0 * BATCH))
    out_ref[...] = jnp.full((1, 1), 0.0, jnp.float32) + loss


def kernel(h, r, pos_t, neg_t, entity_embed, relation_embed):
    ent2 = _make_pack(entity_embed.shape[0])(entity_embed.T)
    rel2 = _make_pack(relation_embed.shape[0])(relation_embed.T)
    delta, l2p = _sc_call(h, r, pos_t, neg_t, ent2, rel2)
    out = pl.pallas_call(
        _tc_body,
        out_shape=jax.ShapeDtypeStruct((1, 1), jnp.float32),
    )(delta, l2p)
    return out[0, 0]


# speedup vs baseline: 4.6625x; 4.6625x over previous
import jax
import jax.numpy as jnp
from jax import lax
from jax.experimental import pallas as pl
from jax.experimental.pallas import tpu as pltpu
from jax.experimental.pallas import tpu_sc as plsc

EMBED = 64
BATCH = 16384
LAM = 1e-05

NC = 2
NS = 16
NW = NC * NS
PW = BATCH // NW
CH = 128
NCH = PW // CH


def _sc_body(h_hbm, r_hbm, p_hbm, n_hbm, ent_hbm, rel_hbm,
             delta_hbm, l2_hbm,
             hidx, ridx, pidx, nidx,
             hbuf, rbuf, pbuf, nbuf,
             dout, l2v, sem):
    wid = lax.axis_index("s") * NC + lax.axis_index("c")
    base = wid * PW

    l2 = jnp.zeros((16,), jnp.float32)
    for c in range(NCH):
        row0 = base + c * CH
        pltpu.sync_copy(h_hbm.at[pl.ds(row0, CH)], hidx.at[c])
        pltpu.sync_copy(r_hbm.at[pl.ds(row0, CH)], ridx.at[c])
        pltpu.sync_copy(p_hbm.at[pl.ds(row0, CH)], pidx.at[c])
        pltpu.sync_copy(n_hbm.at[pl.ds(row0, CH)], nidx.at[c])

        def fire(g, carry):
            sl16 = pl.ds(16 * g, 16)
            hv16 = hidx[c, sl16]
            rv16 = ridx[c, sl16]
            pv16 = pidx[c, sl16]
            nv16 = nidx[c, sl16]
            for l in range(16):
                i = 16 * g + l
                he = hv16[l]
                re = rv16[l]
                pe = pv16[l]
                ne = nv16[l]
                dst = (i >> 3, i & 7)
                pltpu.make_async_copy(ent_hbm.at[he >> 3, he & 7],
                                      hbuf.at[dst[0], dst[1]], sem).start()
                pltpu.make_async_copy(rel_hbm.at[re >> 3, re & 7],
                                      rbuf.at[dst[0], dst[1]], sem).start()
                pltpu.make_async_copy(ent_hbm.at[pe >> 3, pe & 7],
                                      pbuf.at[dst[0], dst[1]], sem).start()
                pltpu.make_async_copy(ent_hbm.at[ne >> 3, ne & 7],
                                      nbuf.at[dst[0], dst[1]], sem).start()
            return carry

        lax.fori_loop(0, CH // 16, fire, 0)
        pltpu.make_async_copy(ent_hbm.at[pl.ds(0, CH // 8)], hbuf, sem).wait()
        pltpu.make_async_copy(ent_hbm.at[pl.ds(0, CH // 8)], rbuf, sem).wait()
        pltpu.make_async_copy(ent_hbm.at[pl.ds(0, CH // 8)], pbuf, sem).wait()
        pltpu.make_async_copy(ent_hbm.at[pl.ds(0, CH // 8)], nbuf, sem).wait()

        def row_body(i, l2c):
            a = i >> 3
            b = i & 7
            dl = jnp.zeros((16,), jnp.float32)
            for d in range(EMBED // 16):
                sl = pl.ds(16 * d, 16)
                hv = hbuf[a, b, sl]
                rv = rbuf[a, b, sl]
                pv = pbuf[a, b, sl]
                nv = nbuf[a, b, sl]
                s = hv + rv
                dp = s - pv
                dn = s - nv
                dl = dl + (dp * dp - dn * dn)
                l2c = l2c + hv * hv + rv * rv + pv * pv + nv * nv
            dout[i, :] = dl
            return l2c

        l2 = lax.fori_loop(0, CH, row_body, l2)
        pltpu.sync_copy(dout, delta_hbm.at[pl.ds(row0, CH)])

    l2v[...] = l2
    pltpu.sync_copy(l2v, l2_hbm.at[wid])


_sc_call = pl.kernel(
    _sc_body,
    out_type=[
        jax.ShapeDtypeStruct((BATCH, 16), jnp.float32),
        jax.ShapeDtypeStruct((NW, 16), jnp.float32),
    ],
    mesh=plsc.VectorSubcoreMesh(core_axis_name="c", subcore_axis_name="s"),
    scratch_types=[
        pltpu.VMEM((NCH, CH), jnp.int32),
        pltpu.VMEM((NCH, CH), jnp.int32),
        pltpu.VMEM((NCH, CH), jnp.int32),
        pltpu.VMEM((NCH, CH), jnp.int32),
        pltpu.VMEM((CH // 8, 8, EMBED), jnp.float32),
        pltpu.VMEM((CH // 8, 8, EMBED), jnp.float32),
        pltpu.VMEM((CH // 8, 8, EMBED), jnp.float32),
        pltpu.VMEM((CH // 8, 8, EMBED), jnp.float32),
        pltpu.VMEM((CH, 16), jnp.float32),
        pltpu.VMEM((16,), jnp.float32),
        pltpu.SemaphoreType.DMA,
    ],
)


def _tc_body(x_ref, l2_ref, out_ref):
    x = x_ref[...]
    g = lax.broadcasted_iota(jnp.int32, (128, 8), 0) // 16
    c = lax.broadcasted_iota(jnp.int32, (128, 8), 1)
    m = (g == c).astype(jnp.float32)
    y = lax.dot_general(x, m, (((1,), (0,)), ((), ())),
                        preferred_element_type=jnp.float32)
    sp = jnp.maximum(y, 0.0) + jnp.log1p(jnp.exp(-jnp.abs(y)))
    l2tot = jnp.sum(l2_ref[...])
    loss = jnp.sum(sp) / BATCH + LAM * (l2tot / (2.0 * BATCH))
    out_ref[...] = jnp.full((1, 1), 0.0, jnp.float32) + loss


def kernel(h, r, pos_t, neg_t, entity_embed, relation_embed):
    ent3 = entity_embed.reshape(-1, 8, EMBED)
    rel3 = relation_embed.reshape(-1, 8, EMBED)
    delta, l2p = _sc_call(h, r, pos_t, neg_t, ent3, rel3)
    x = delta.reshape(BATCH // 8, 128)
    l2x = l2p.reshape(NW * 16 // 128, 128)
    out = pl.pallas_call(
        _tc_body,
        out_shape=jax.ShapeDtypeStruct((1, 1), jnp.float32),
    )(x, l2x)
    return out[0, 0]


# speedup vs baseline: 4.7589x; 1.0207x over previous
import jax
import jax.numpy as jnp
from jax import lax
from jax.experimental import pallas as pl
from jax.experimental.pallas import tpu as pltpu
from jax.experimental.pallas import tpu_sc as plsc

EMBED = 64
BATCH = 16384
LAM = 1e-05

NC = 2
NS = 16
NW = NC * NS
PW = BATCH // NW
CH = 64
NCH = PW // CH


def _sc_body(h_hbm, r_hbm, p_hbm, n_hbm, ent_hbm, rel_hbm,
             delta_hbm, l2_hbm,
             hidx, ridx, pidx, nidx,
             hbuf, rbuf, pbuf, nbuf,
             dout, l2v, sem0, sem1):
    wid = lax.axis_index("s") * NC + lax.axis_index("c")
    base = wid * PW
    sems = (sem0, sem1)

    def fire(c):
        k = c % 2
        sem = sems[k]
        row0 = base + c * CH
        pltpu.sync_copy(h_hbm.at[pl.ds(row0, CH)], hidx.at[c])
        pltpu.sync_copy(r_hbm.at[pl.ds(row0, CH)], ridx.at[c])
        pltpu.sync_copy(p_hbm.at[pl.ds(row0, CH)], pidx.at[c])
        pltpu.sync_copy(n_hbm.at[pl.ds(row0, CH)], nidx.at[c])

        def body(g, carry):
            sl16 = pl.ds(16 * g, 16)
            hv16 = hidx[c, sl16]
            rv16 = ridx[c, sl16]
            pv16 = pidx[c, sl16]
            nv16 = nidx[c, sl16]
            for l in range(16):
                i = 16 * g + l
                he = hv16[l]
                re = rv16[l]
                pe = pv16[l]
                ne = nv16[l]
                a = i >> 3
                b = i & 7
                pltpu.make_async_copy(ent_hbm.at[he >> 3, he & 7],
                                      hbuf.at[k, a, b], sem).start()
                pltpu.make_async_copy(rel_hbm.at[re >> 3, re & 7],
                                      rbuf.at[k, a, b], sem).start()
                pltpu.make_async_copy(ent_hbm.at[pe >> 3, pe & 7],
                                      pbuf.at[k, a, b], sem).start()
                pltpu.make_async_copy(ent_hbm.at[ne >> 3, ne & 7],
                                      nbuf.at[k, a, b], sem).start()
            return carry

        lax.fori_loop(0, CH // 16, body, 0)

    def compute(c, l2):
        k = c % 2
        sem = sems[k]
        pltpu.make_async_copy(ent_hbm.at[pl.ds(0, CH // 8)], hbuf.at[k], sem).wait()
        pltpu.make_async_copy(ent_hbm.at[pl.ds(0, CH // 8)], rbuf.at[k], sem).wait()
        pltpu.make_async_copy(ent_hbm.at[pl.ds(0, CH // 8)], pbuf.at[k], sem).wait()
        pltpu.make_async_copy(ent_hbm.at[pl.ds(0, CH // 8)], nbuf.at[k], sem).wait()

        def row_body(i, l2c):
            a = i >> 3
            b = i & 7
            dl = jnp.zeros((16,), jnp.float32)
            for d in range(EMBED // 16):
                sl = pl.ds(16 * d, 16)
                hv = hbuf[k, a, b, sl]
                rv = rbuf[k, a, b, sl]
                pv = pbuf[k, a, b, sl]
                nv = nbuf[k, a, b, sl]
                s = hv + rv
                dp = s - pv
                dn = s - nv
                dl = dl + (dp * dp - dn * dn)
                l2c = l2c + hv * hv + rv * rv + pv * pv + nv * nv
            dout[i >> 3, pl.ds(16 * (i & 7), 16)] = dl
            return l2c

        l2 = lax.fori_loop(0, CH, row_body, l2)
        pltpu.sync_copy(dout, delta_hbm.at[pl.ds(wid * (PW // 8) + c * (CH // 8), CH // 8)])
        return l2

    fire(0)
    l2 = jnp.zeros((16,), jnp.float32)
    for c in range(NCH):
        if c + 1 < NCH:
            fire(c + 1)
        l2 = compute(c, l2)

    l2v[...] = l2
    pltpu.sync_copy(l2v, l2_hbm.at[wid >> 3, pl.ds(16 * (wid & 7), 16)])


_sc_call = pl.kernel(
    _sc_body,
    out_type=[
        jax.ShapeDtypeStruct((BATCH // 8, 128), jnp.float32),
        jax.ShapeDtypeStruct((NW // 8, 128), jnp.float32),
    ],
    mesh=plsc.VectorSubcoreMesh(core_axis_name="c", subcore_axis_name="s"),
    scratch_types=[
        pltpu.VMEM((NCH, CH), jnp.int32),
        pltpu.VMEM((NCH, CH), jnp.int32),
        pltpu.VMEM((NCH, CH), jnp.int32),
        pltpu.VMEM((NCH, CH), jnp.int32),
        pltpu.VMEM((2, CH // 8, 8, EMBED), jnp.float32),
        pltpu.VMEM((2, CH // 8, 8, EMBED), jnp.float32),
        pltpu.VMEM((2, CH // 8, 8, EMBED), jnp.float32),
        pltpu.VMEM((2, CH // 8, 8, EMBED), jnp.float32),
        pltpu.VMEM((CH // 8, 128), jnp.float32),
        pltpu.VMEM((16,), jnp.float32),
        pltpu.SemaphoreType.DMA,
        pltpu.SemaphoreType.DMA,
    ],
)


def _tc_body(x_ref, l2_ref, out_ref):
    x = x_ref[...]
    g = lax.broadcasted_iota(jnp.int32, (128, 8), 0) // 16
    c = lax.broadcasted_iota(jnp.int32, (128, 8), 1)
    m = (g == c).astype(jnp.float32)
    y = lax.dot_general(x, m, (((1,), (0,)), ((), ())),
                        preferred_element_type=jnp.float32)
    sp = jnp.maximum(y, 0.0) + jnp.log1p(jnp.exp(-jnp.abs(y)))
    l2tot = jnp.sum(l2_ref[...])
    loss = jnp.sum(sp) / BATCH + LAM * (l2tot / (2.0 * BATCH))
    out_ref[...] = jnp.full((1, 1), 0.0, jnp.float32) + loss


def kernel(h, r, pos_t, neg_t, entity_embed, relation_embed):
    ent3 = entity_embed.reshape(-1, 8, EMBED)
    rel3 = relation_embed.reshape(-1, 8, EMBED)
    delta, l2p = _sc_call(h, r, pos_t, neg_t, ent3, rel3)
    out = pl.pallas_call(
        _tc_body,
        out_shape=jax.ShapeDtypeStruct((1, 1), jnp.float32),
    )(delta, l2p)
    return out[0, 0]
